# Initial kernel scaffold; baseline (speedup 1.0000x reference)
import jax
import jax.numpy as jnp
from jax.experimental import pallas as pl


def kernel(med_ids, demo_features, embed_table, W1, b1, W2, b2):
    raise NotImplementedError("write your pallas kernel here")


# speedup vs baseline: 5.1690x; 5.1690x over previous
import functools

import jax
import jax.numpy as jnp
from jax import lax
from jax.experimental import pallas as pl
from jax.experimental.pallas import tpu as pltpu
from jax.experimental.pallas import tpu_sc as plsc

_B = 4096
_H = 50
_D = 32
_DEMO = 10
_HID = 64

_NC, _NS = 2, 16
_NW = _NC * _NS
_SPW = _B // _NW
_SPC = 2
_CHUNKS = _SPW // _SPC
_GIDX = _SPC * _H
_GPAD = 104


def _sc_gather_pool(ids_p, table):
    mesh = plsc.VectorSubcoreMesh(core_axis_name="c", subcore_axis_name="s")

    @functools.partial(
        pl.kernel,
        mesh=mesh,
        out_type=jax.ShapeDtypeStruct((_B, _D), jnp.float32),
        scratch_types=[
            pltpu.VMEM((_CHUNKS, _GPAD), jnp.int32),
            pltpu.VMEM((_GPAD, _D), jnp.float32),
            pltpu.VMEM((_GPAD, _D), jnp.float32),
            pltpu.VMEM((_SPW, _D), jnp.float32),
            pltpu.SemaphoreType.DMA,
            pltpu.SemaphoreType.DMA,
        ],
        compiler_params=pltpu.CompilerParams(use_tc_tiling_on_sc=False),
    )
    def k(ids_hbm, table_hbm, out_hbm, idx_v, rows0, rows1, acc_v, sem0, sem1):
        wid = lax.axis_index("s") * _NC + lax.axis_index("c")
        pltpu.sync_copy(ids_hbm.at[wid], idx_v)

        bufs = (rows0, rows1)
        sems = (sem0, sem1)

        def fire(c, b):
            pltpu.async_copy(table_hbm.at[idx_v.at[c]], bufs[b], sems[b])

        fire(0, 0)
        fire(1, 1)

        @pl.loop(0, _CHUNKS, step=2)
        def _(j):
            for b in range(2):
                c = j + b
                pltpu.make_async_copy(
                    table_hbm.at[idx_v.at[c]], bufs[b], sems[b]).wait()
                buf = bufs[b]
                for s in range(_SPC):
                    for l in range(_D // 16):
                        v = buf[s * _H, pl.ds(16 * l, 16)]
                        for r in range(1, _H):
                            v = v + buf[s * _H + r, pl.ds(16 * l, 16)]
                        acc_v[c * _SPC + s, pl.ds(16 * l, 16)] = v

                @pl.when(c + 2 < _CHUNKS)
                def _():
                    fire(c + 2, b)

        pltpu.sync_copy(acc_v, out_hbm.at[pl.ds(wid * _SPW, _SPW)])

    return k(ids_p, table)


def _tc_mlp(pooled, demo, w1, b1, w2, b2):
    def body(mv_ref, demo_ref, w1_ref, b1_ref, w2_ref, b2_ref, out_ref):
        mv = mv_ref[...] * (1.0 / _H)
        w1m = w1_ref[...]
        x1 = lax.dot_general(mv, w1m[:, :_D], (((1,), (1,)), ((), ())),
                             preferred_element_type=jnp.float32)
        x2 = lax.dot_general(demo_ref[...], w1m[:, _D:],
                             (((1,), (1,)), ((), ())),
                             preferred_element_type=jnp.float32)
        h = jax.nn.relu(x1 + x2 + b1_ref[...])
        o = jnp.sum(h * w2_ref[...], axis=1, keepdims=True) + b2_ref[0]
        out_ref[...] = jax.nn.sigmoid(o)

    return pl.pallas_call(
        body,
        in_specs=[pl.BlockSpec(memory_space=pltpu.VMEM)] * 5
        + [pl.BlockSpec(memory_space=pltpu.SMEM)],
        out_specs=pl.BlockSpec(memory_space=pltpu.VMEM),
        out_shape=jax.ShapeDtypeStruct((_B, 1), jnp.float32),
    )(pooled, demo, w1, b1.reshape(1, _HID), w2, b2)


def kernel(med_ids, demo_features, embed_table, W1, b1, W2, b2):
    ids = med_ids.astype(jnp.int32).reshape(_NW, _CHUNKS, _GIDX)
    ids_p = jnp.pad(ids, ((0, 0), (0, 0), (0, _GPAD - _GIDX)))
    pooled = _sc_gather_pool(ids_p, embed_table)
    return _tc_mlp(pooled, demo_features, W1, b1, W2, b2)
